# Initial kernel scaffold; baseline (speedup 1.0000x reference)
import jax
import jax.numpy as jnp
from jax.experimental import pallas as pl


def kernel(mem, node_idxs, values):
    raise NotImplementedError("write your pallas kernel here")


# speedup vs baseline: 16.8241x; 16.8241x over previous
import functools

import jax
import jax.numpy as jnp
from jax import lax
from jax.experimental import pallas as pl
from jax.experimental.pallas import tpu as pltpu
from jax.experimental.pallas import tpu_sc as plsc

N_NODES = 1_000_000
MEM_DIM = 64
BATCH = 16384

LANES = 16
NUM_CORES = 2
NUM_SUBCORES = 16
NUM_WORKERS = NUM_CORES * NUM_SUBCORES

NODES_PER_TILE = 65536
VECS = BATCH // LANES
OUT_PER_W = BATCH // NUM_WORKERS
CHUNK = 128
NCHUNKS = OUT_PER_W // CHUNK
N_CORRECTION_SWEEPS = 3


def _body(idx_hbm, values_hbm, out_hbm, p_hbm,
          idx_v, p_local, idx_c, w_c, rows_c, sem):
    cid = lax.axis_index("c")
    sid = lax.axis_index("s")
    base = sid * NODES_PER_TILE
    lanes = lax.iota(jnp.int32, LANES)

    pltpu.sync_copy(idx_hbm, idx_v)

    def sweep(v, do_correct):
        off = pl.multiple_of(v * LANES, LANES)
        iv = idx_v[pl.ds(off, LANES)]
        jv = lanes + v * LANES
        m = (iv >= base) & (iv < base + NODES_PER_TILE)
        loc = jnp.where(m, iv - base, 0)
        if do_correct:
            g = plsc.load_gather(p_local, [loc], mask=m)
            m = m & (jv > g)
        plsc.store_scatter(p_local, [loc], jv, mask=m)
        return 0

    lax.fori_loop(0, VECS, lambda v, c: sweep(v, False), 0)
    for _ in range(N_CORRECTION_SWEEPS):
        lax.fori_loop(0, VECS, lambda v, c: sweep(v, True), 0)

    pltpu.sync_copy(p_local, p_hbm.at[pl.ds(base, NODES_PER_TILE)])
    plsc.subcore_barrier()

    wid = sid * NUM_CORES + cid
    for c in range(NCHUNKS):
        start = pl.multiple_of(wid * OUT_PER_W + c * CHUNK, CHUNK)
        pltpu.sync_copy(idx_hbm.at[pl.ds(start, CHUNK)], idx_c)
        pltpu.async_copy(p_hbm.at[idx_c], w_c, sem).wait()
        pltpu.async_copy(values_hbm.at[w_c], rows_c, sem).wait()
        pltpu.sync_copy(rows_c, out_hbm.at[pl.ds(start, CHUNK)])


@functools.partial(jax.jit, donate_argnums=())
def _scatter_gather(node_idxs, values):
    mesh = plsc.VectorSubcoreMesh(
        core_axis_name="c", subcore_axis_name="s")
    out, _ = pl.kernel(
        _body,
        out_type=(
            jax.ShapeDtypeStruct((BATCH, MEM_DIM), jnp.float32),
            jax.ShapeDtypeStruct((N_NODES + 48576, ), jnp.int32),
        ),
        mesh=mesh,
        scratch_types=[
            pltpu.VMEM((BATCH,), jnp.int32),
            pltpu.VMEM((NODES_PER_TILE,), jnp.int32),
            pltpu.VMEM((CHUNK,), jnp.int32),
            pltpu.VMEM((CHUNK,), jnp.int32),
            pltpu.VMEM((CHUNK, MEM_DIM), jnp.float32),
            pltpu.SemaphoreType.DMA,
        ],
        compiler_params=pltpu.CompilerParams(
            needs_layout_passes=False, use_tc_tiling_on_sc=False),
    )(node_idxs, values)
    return out


def kernel(mem, node_idxs, values):
    del mem
    return _scatter_gather(node_idxs, values)


# speedup vs baseline: 23.8919x; 1.4201x over previous
import functools

import jax
import jax.numpy as jnp
from jax import lax
from jax.experimental import pallas as pl
from jax.experimental.pallas import tpu as pltpu
from jax.experimental.pallas import tpu_sc as plsc

N_NODES = 1_000_000
MEM_DIM = 64
BATCH = 16384

LANES = 16
NUM_CORES = 2
NUM_SUBCORES = 16
NUM_WORKERS = NUM_CORES * NUM_SUBCORES

NODES_PER_TILE = 65536
P_SIZE = NODES_PER_TILE * NUM_SUBCORES
VECS = BATCH // LANES
UNROLL = 4
OUT_PER_W = BATCH // NUM_WORKERS
CHUNK = 128
NCHUNKS = OUT_PER_W // CHUNK


def _body(idx_hbm, values_hbm, out_hbm, p_hbm,
          idx_v, p_local, idx_c, w_v, rows_v, sem):
    cid = lax.axis_index("c")
    sid = lax.axis_index("s")
    base = sid * NODES_PER_TILE
    lanes = lax.iota(jnp.int32, LANES)
    nbr_perm = jnp.minimum(lanes + 1, LANES - 1)
    last_lane = lanes == LANES - 1

    pltpu.sync_copy(idx_hbm, idx_v)

    def sweep_one(v):
        off = pl.multiple_of(v * LANES, LANES)
        iv = idx_v[pl.ds(off, LANES)]
        key = (iv << 4) | lanes
        key_s = lax.sort(key)
        idx_s = key_s >> 4
        j_s = (key_s & (LANES - 1)) + v * LANES
        nbr = key_s[nbr_perm] >> 4
        m = (idx_s >= base) & (idx_s < base + NODES_PER_TILE)
        m = m & ((idx_s != nbr) | last_lane)
        loc = jnp.where(m, idx_s - base, 0)
        plsc.store_scatter(p_local, [loc], j_s, mask=m)

    def sweep(u, carry):
        for k in range(UNROLL):
            sweep_one(u * UNROLL + k)
        return carry

    lax.fori_loop(0, VECS // UNROLL, sweep, 0)

    pltpu.sync_copy(p_local, p_hbm.at[pl.ds(base, NODES_PER_TILE)])
    plsc.subcore_barrier()

    wid = sid * NUM_CORES + cid
    obase = wid * OUT_PER_W
    for c in range(NCHUNKS):
        start = pl.multiple_of(obase + c * CHUNK, CHUNK)
        pltpu.sync_copy(idx_hbm.at[pl.ds(start, CHUNK)], idx_c)
        pltpu.async_copy(p_hbm.at[idx_c], w_v.at[c], sem).wait()
        pltpu.async_copy(values_hbm.at[w_v.at[c]], rows_v.at[c], sem).wait()
        pltpu.sync_copy(rows_v.at[c], out_hbm.at[pl.ds(start, CHUNK)])


@jax.jit
def _scatter_gather(node_idxs, values):
    mesh = plsc.VectorSubcoreMesh(
        core_axis_name="c", subcore_axis_name="s")
    out, _ = pl.kernel(
        _body,
        out_type=(
            jax.ShapeDtypeStruct((BATCH, MEM_DIM), jnp.float32),
            jax.ShapeDtypeStruct((P_SIZE,), jnp.int32),
        ),
        mesh=mesh,
        scratch_types=[
            pltpu.VMEM((BATCH,), jnp.int32),
            pltpu.VMEM((NODES_PER_TILE,), jnp.int32),
            pltpu.VMEM((CHUNK,), jnp.int32),
            pltpu.VMEM((NCHUNKS, CHUNK), jnp.int32),
            pltpu.VMEM((NCHUNKS, CHUNK, MEM_DIM), jnp.float32),
            pltpu.SemaphoreType.DMA,
        ],
        compiler_params=pltpu.CompilerParams(
            needs_layout_passes=False, use_tc_tiling_on_sc=False),
    )(node_idxs, values)
    return out


def kernel(mem, node_idxs, values):
    del mem
    return _scatter_gather(node_idxs, values)


# speedup vs baseline: 25.1643x; 1.0533x over previous
import jax
import jax.numpy as jnp
from jax import lax
from jax.experimental import pallas as pl
from jax.experimental.pallas import tpu as pltpu
from jax.experimental.pallas import tpu_sc as plsc

N_NODES = 1_000_000
MEM_DIM = 64
BATCH = 16384

LANES = 16
NUM_CORES = 2
NUM_SUBCORES = 16
NUM_WORKERS = NUM_CORES * NUM_SUBCORES

NODES_PER_TILE = 65536
P_SIZE = NODES_PER_TILE * NUM_SUBCORES
VECS = BATCH // LANES
UNROLL = 4
OUT_PER_W = BATCH // NUM_WORKERS
CHUNK = 128
NCHUNKS = OUT_PER_W // CHUNK


def _body(idx_hbm, values_hbm, out_hbm, p_hbm,
          idx_v, p_local, idx_c, w_v, rows_v, wsem, rsem, osem):
    cid = lax.axis_index("c")
    sid = lax.axis_index("s")
    base = sid * NODES_PER_TILE
    lanes = lax.iota(jnp.int32, LANES)
    nbr_perm = jnp.minimum(lanes + 1, LANES - 1)
    last_lane = lanes == LANES - 1

    pltpu.sync_copy(idx_hbm, idx_v)

    def sweep_one(v):
        off = pl.multiple_of(v * LANES, LANES)
        iv = idx_v[pl.ds(off, LANES)]
        key = (iv << 4) | lanes
        key_s = lax.sort(key)
        idx_s = key_s >> 4
        j_s = (key_s & (LANES - 1)) + v * LANES
        nbr = key_s[nbr_perm] >> 4
        m = (idx_s >= base) & (idx_s < base + NODES_PER_TILE)
        m = m & ((idx_s != nbr) | last_lane)
        loc = jnp.where(m, idx_s - base, 0)
        plsc.store_scatter(p_local, [loc], j_s, mask=m)

    def sweep(u, carry):
        for k in range(UNROLL):
            sweep_one(u * UNROLL + k)
        return carry

    lax.fori_loop(0, VECS // UNROLL, sweep, 0)

    pltpu.sync_copy(p_local, p_hbm.at[pl.ds(base, NODES_PER_TILE)])
    plsc.subcore_barrier()

    wid = sid * NUM_CORES + cid
    obase = wid * OUT_PER_W
    starts = [pl.multiple_of(obase + c * CHUNK, CHUNK) for c in range(NCHUNKS)]

    pltpu.sync_copy(idx_hbm.at[pl.ds(starts[0], CHUNK)], idx_c.at[0])
    wd = pltpu.async_copy(p_hbm.at[idx_c.at[0]], w_v.at[0], wsem)
    rd = od = None
    for c in range(NCHUNKS):
        b = c % 2
        if c + 1 < NCHUNKS:
            pltpu.sync_copy(idx_hbm.at[pl.ds(starts[c + 1], CHUNK)],
                            idx_c.at[1 - b])
        wd.wait()
        if rd is not None:
            rd.wait()
            if od is not None:
                od.wait()
            od = pltpu.async_copy(rows_v.at[1 - b],
                                  out_hbm.at[pl.ds(starts[c - 1], CHUNK)],
                                  osem)
        if c + 1 < NCHUNKS:
            wd = pltpu.async_copy(p_hbm.at[idx_c.at[1 - b]],
                                  w_v.at[1 - b], wsem)
        rd = pltpu.async_copy(values_hbm.at[w_v.at[b]], rows_v.at[b], rsem)
    rd.wait()
    if od is not None:
        od.wait()
    pltpu.async_copy(rows_v.at[(NCHUNKS - 1) % 2],
                     out_hbm.at[pl.ds(starts[NCHUNKS - 1], CHUNK)],
                     osem).wait()


@jax.jit
def _scatter_gather(node_idxs, values):
    mesh = plsc.VectorSubcoreMesh(
        core_axis_name="c", subcore_axis_name="s")
    out, _ = pl.kernel(
        _body,
        out_type=(
            jax.ShapeDtypeStruct((BATCH, MEM_DIM), jnp.float32),
            jax.ShapeDtypeStruct((P_SIZE,), jnp.int32),
        ),
        mesh=mesh,
        scratch_types=[
            pltpu.VMEM((BATCH,), jnp.int32),
            pltpu.VMEM((NODES_PER_TILE,), jnp.int32),
            pltpu.VMEM((2, CHUNK), jnp.int32),
            pltpu.VMEM((2, CHUNK), jnp.int32),
            pltpu.VMEM((2, CHUNK, MEM_DIM), jnp.float32),
            pltpu.SemaphoreType.DMA,
            pltpu.SemaphoreType.DMA,
            pltpu.SemaphoreType.DMA,
        ],
        compiler_params=pltpu.CompilerParams(
            needs_layout_passes=False, use_tc_tiling_on_sc=False),
    )(node_idxs, values)
    return out


def kernel(mem, node_idxs, values):
    del mem
    return _scatter_gather(node_idxs, values)


# speedup vs baseline: 30.2621x; 1.2026x over previous
import jax
import jax.numpy as jnp
from jax import lax
from jax.experimental import pallas as pl
from jax.experimental.pallas import tpu as pltpu
from jax.experimental.pallas import tpu_sc as plsc

N_NODES = 1_000_000
MEM_DIM = 64
BATCH = 16384

LANES = 16
NUM_CORES = 2
NUM_SUBCORES = 16
NUM_WORKERS = NUM_CORES * NUM_SUBCORES

NODES_PER_TILE = 65536
P_SIZE = NODES_PER_TILE * NUM_SUBCORES
VECS = BATCH // LANES
UNROLL = 4
OUT_PER_W = BATCH // NUM_WORKERS
CHUNK = 128
NCHUNKS = OUT_PER_W // CHUNK


def _body(idx_hbm, values_hbm, out_hbm, p_hbm,
          idx_v, p_local, idx_c, w_v, rows_v, wsem, rsem, osem):
    cid = lax.axis_index("c")
    sid = lax.axis_index("s")
    base = sid * NODES_PER_TILE
    lanes = lax.iota(jnp.int32, LANES)
    nbr_perm = jnp.minimum(lanes + 1, LANES - 1)
    last_lane = lanes == LANES - 1

    pltpu.sync_copy(idx_hbm, idx_v)

    def sweep(u, carry):
        vs = [u * UNROLL + k for k in range(UNROLL)]
        keys = []
        for v in vs:
            off = pl.multiple_of(v * LANES, LANES)
            iv = idx_v[pl.ds(off, LANES)]
            keys.append(lax.sort((iv << 4) | lanes))
        for v, key_s in zip(vs, keys):
            idx_s = key_s >> 4
            j_s = (key_s & (LANES - 1)) + v * LANES
            nbr = key_s[nbr_perm] >> 4
            m = (idx_s >= base) & (idx_s < base + NODES_PER_TILE)
            m = m & ((idx_s != nbr) | last_lane)
            loc = jnp.where(m, idx_s - base, 0)
            plsc.store_scatter(p_local, [loc], j_s, mask=m)
        return carry

    lax.fori_loop(0, VECS // UNROLL, sweep, 0)

    pltpu.sync_copy(p_local, p_hbm.at[pl.ds(base, NODES_PER_TILE)])
    plsc.subcore_barrier()

    wid = sid * NUM_CORES + cid
    obase = wid * OUT_PER_W
    starts = [pl.multiple_of(obase + c * CHUNK, CHUNK) for c in range(NCHUNKS)]

    pltpu.sync_copy(idx_hbm.at[pl.ds(starts[0], CHUNK)], idx_c.at[0])
    wd = pltpu.async_copy(p_hbm.at[idx_c.at[0]], w_v.at[0], wsem)
    rd = od = None
    for c in range(NCHUNKS):
        b = c % 2
        if c + 1 < NCHUNKS:
            pltpu.sync_copy(idx_hbm.at[pl.ds(starts[c + 1], CHUNK)],
                            idx_c.at[1 - b])
        wd.wait()
        if rd is not None:
            rd.wait()
            if od is not None:
                od.wait()
            od = pltpu.async_copy(rows_v.at[1 - b],
                                  out_hbm.at[pl.ds(starts[c - 1], CHUNK)],
                                  osem)
        if c + 1 < NCHUNKS:
            wd = pltpu.async_copy(p_hbm.at[idx_c.at[1 - b]],
                                  w_v.at[1 - b], wsem)
        rd = pltpu.async_copy(values_hbm.at[w_v.at[b]], rows_v.at[b], rsem)
    rd.wait()
    if od is not None:
        od.wait()
    pltpu.async_copy(rows_v.at[(NCHUNKS - 1) % 2],
                     out_hbm.at[pl.ds(starts[NCHUNKS - 1], CHUNK)],
                     osem).wait()


@jax.jit
def _scatter_gather(node_idxs, values):
    mesh = plsc.VectorSubcoreMesh(
        core_axis_name="c", subcore_axis_name="s")
    out, _ = pl.kernel(
        _body,
        out_type=(
            jax.ShapeDtypeStruct((BATCH, MEM_DIM), jnp.float32),
            jax.ShapeDtypeStruct((P_SIZE,), jnp.int32),
        ),
        mesh=mesh,
        scratch_types=[
            pltpu.VMEM((BATCH,), jnp.int32),
            pltpu.VMEM((NODES_PER_TILE,), jnp.int32),
            pltpu.VMEM((2, CHUNK), jnp.int32),
            pltpu.VMEM((2, CHUNK), jnp.int32),
            pltpu.VMEM((2, CHUNK, MEM_DIM), jnp.float32),
            pltpu.SemaphoreType.DMA,
            pltpu.SemaphoreType.DMA,
            pltpu.SemaphoreType.DMA,
        ],
        compiler_params=pltpu.CompilerParams(
            needs_layout_passes=False, use_tc_tiling_on_sc=False),
    )(node_idxs, values)
    return out


def kernel(mem, node_idxs, values):
    del mem
    return _scatter_gather(node_idxs, values)
